# Initial kernel scaffold; baseline (speedup 1.0000x reference)
import jax
import jax.numpy as jnp
from jax.experimental import pallas as pl


def kernel(token_ids, embeddings):
    raise NotImplementedError("write your pallas kernel here")


# speedup vs baseline: 4.8105x; 4.8105x over previous
import functools

import jax
import jax.numpy as jnp
from jax import lax
from jax.experimental import pallas as pl
from jax.experimental.pallas import tpu as pltpu
from jax.experimental.pallas import tpu_sc as plsc

_B = 16384 * 200
_D = 32
_NW = 32
_BPW = _B // _NW
_K = 8
_CHUNK = _K * 128
_NCHUNK = _BPW // _CHUNK

_mesh = plsc.VectorSubcoreMesh(core_axis_name="c", subcore_axis_name="s")


@functools.partial(
    pl.kernel,
    mesh=_mesh,
    compiler_params=pltpu.CompilerParams(use_tc_tiling_on_sc=False),
    out_type=jax.ShapeDtypeStruct((_B, _D), jnp.float32),
    scratch_types=[
        pltpu.VMEM((_K, 128), jnp.int32),
        pltpu.VMEM((_CHUNK, _D), jnp.float32),
        pltpu.SemaphoreType.DMA,
    ],
)
def _gather_kernel(idx_hbm, table_hbm, out_hbm, idx_v, rows_v, sem):
    wid = lax.axis_index("s") * 2 + lax.axis_index("c")
    base128 = wid * (_BPW // 128)

    def body(i, carry):
        rbase = base128 + i * _K
        pltpu.sync_copy(idx_hbm.at[pl.ds(rbase, _K)], idx_v)
        copies = [
            pltpu.async_copy(
                table_hbm.at[idx_v.at[j]],
                rows_v.at[pl.ds(j * 128, 128)],
                sem,
            )
            for j in range(_K)
        ]
        for c in copies:
            c.wait()
        pltpu.sync_copy(rows_v, out_hbm.at[pl.ds(rbase * 128, _CHUNK)])
        return carry

    lax.fori_loop(0, _NCHUNK, body, 0)


def kernel(token_ids, embeddings):
    ids = token_ids.reshape(_B // 128, 128).astype(jnp.int32)
    out = _gather_kernel(ids, embeddings)
    return out.reshape(*token_ids.shape, _D)


# speedup vs baseline: 5.0476x; 1.0493x over previous
import functools

import jax
import jax.numpy as jnp
from jax import lax
from jax.experimental import pallas as pl
from jax.experimental.pallas import tpu as pltpu
from jax.experimental.pallas import tpu_sc as plsc

_B = 16384 * 200
_D = 32
_NW = 32
_BPW = _B // _NW
_K = 8
_CHUNK = _K * 128
_NCHUNK = _BPW // _CHUNK

_mesh = plsc.VectorSubcoreMesh(core_axis_name="c", subcore_axis_name="s")


@functools.partial(
    pl.kernel,
    mesh=_mesh,
    compiler_params=pltpu.CompilerParams(use_tc_tiling_on_sc=False),
    out_type=jax.ShapeDtypeStruct((_B, _D), jnp.float32),
    scratch_types=[
        pltpu.VMEM((_K, 128), jnp.int32),
        pltpu.VMEM((_K, 128), jnp.int32),
        pltpu.VMEM((_CHUNK, _D), jnp.float32),
        pltpu.VMEM((_CHUNK, _D), jnp.float32),
        pltpu.SemaphoreType.DMA,
        pltpu.SemaphoreType.DMA,
        pltpu.SemaphoreType.DMA,
        pltpu.SemaphoreType.DMA,
        pltpu.SemaphoreType.DMA,
        pltpu.SemaphoreType.DMA,
    ],
)
def _gather_kernel(idx_hbm, table_hbm, out_hbm,
                   idx0, idx1, rows0, rows1,
                   isem0, isem1, gsem0, gsem1, osem0, osem1):
    idx_vs = (idx0, idx1)
    rows_vs = (rows0, rows1)
    isems = (isem0, isem1)
    gsems = (gsem0, gsem1)
    osems = (osem0, osem1)

    wid = lax.axis_index("s") * 2 + lax.axis_index("c")
    base128 = wid * (_BPW // 128)

    def start_idx(g, s):
        pltpu.async_copy(idx_hbm.at[pl.ds(base128 + g * _K, _K)],
                         idx_vs[s], isems[s])

    def wait_idx(s):
        pltpu.make_async_copy(idx_hbm.at[pl.ds(0, _K)],
                              idx_vs[s], isems[s]).wait()

    def start_gather(s):
        for j in range(_K):
            pltpu.async_copy(table_hbm.at[idx_vs[s].at[j]],
                             rows_vs[s].at[pl.ds(j * 128, 128)], gsems[s])

    def wait_gather(s):
        for j in range(_K):
            pltpu.make_async_copy(table_hbm.at[idx_vs[s].at[j]],
                                  rows_vs[s].at[pl.ds(j * 128, 128)],
                                  gsems[s]).wait()

    def start_store(g, s):
        pltpu.async_copy(rows_vs[s],
                         out_hbm.at[pl.ds((base128 + g * _K) * 128, _CHUNK)],
                         osems[s])

    def wait_store(s):
        pltpu.make_async_copy(rows_vs[s],
                              out_hbm.at[pl.ds(0, _CHUNK)], osems[s]).wait()

    start_idx(0, 0)
    wait_idx(0)
    start_gather(0)
    start_idx(1, 1)
    wait_idx(1)
    start_gather(1)
    wait_gather(0)
    start_store(0, 0)
    start_idx(2, 0)
    wait_idx(0)
    wait_store(0)
    start_gather(0)
    wait_gather(1)
    start_store(1, 1)
    start_idx(3, 1)

    def body(t, carry):
        ga = 2 * t + 3
        wait_idx(1)
        wait_store(1)
        start_gather(1)
        wait_gather(0)
        start_store(ga - 1, 0)
        start_idx(ga + 1, 0)
        wait_idx(0)
        wait_store(0)
        start_gather(0)
        wait_gather(1)
        start_store(ga, 1)
        start_idx(ga + 2, 1)
        return carry

    lax.fori_loop(0, (_NCHUNK - 4) // 2, body, 0)

    wait_idx(1)
    wait_store(1)
    start_gather(1)
    wait_gather(0)
    start_store(_NCHUNK - 2, 0)
    wait_gather(1)
    start_store(_NCHUNK - 1, 1)
    wait_store(0)
    wait_store(1)


def kernel(token_ids, embeddings):
    ids = token_ids.reshape(_B // 128, 128).astype(jnp.int32)
    out = _gather_kernel(ids, embeddings)
    return out.reshape(*token_ids.shape, _D)
